# speedup vs baseline: 20.8778x; 20.8778x over previous
import functools

import jax
import jax.numpy as jnp
from jax import lax
from jax.experimental import pallas as pl
from jax.experimental.pallas import tpu as pltpu
from jax.experimental.pallas import tpu_sc as plsc

NC = 2
NS = 16
NW = NC * NS
CHUNK = 128


def _mesh():
    return plsc.VectorSubcoreMesh(
        core_axis_name="c", subcore_axis_name="s", num_cores=NC,
        num_subcores=NS)


@functools.lru_cache(maxsize=None)
def _build(N, E, D_in, D_out):
    NPAD = ((N + 1023) // 1024) * 1024
    E_PAD = ((E + NW * CHUNK - 1) // (NW * CHUNK)) * (NW * CHUNK)
    KCH = E_PAD // (NW * CHUNK)
    DSTRIPE = NPAD // NS
    RSTRIPE = NPAD // NS

    @functools.partial(
        pl.kernel,
        out_type=jax.ShapeDtypeStruct((2 * NPAD,), jnp.float32),
        mesh=_mesh(),
        scratch_types=[
            pltpu.VMEM((CHUNK,), jnp.int32),
            pltpu.VMEM((CHUNK,), jnp.float32),
            pltpu.VMEM((DSTRIPE,), jnp.float32),
            pltpu.VMEM_SHARED((NPAD,), jnp.float32),
        ],
    )
    def _deg(col_hbm, out_hbm, colv, onesv, zv, acc):
        cid = lax.axis_index("c")
        sid = lax.axis_index("s")
        wid = sid * NC + cid
        for i in range(CHUNK // 16):
            onesv[pl.ds(i * 16, 16)] = jnp.ones((16,), jnp.float32)

        def zbody(i, carry):
            zv[pl.ds(pl.multiple_of(i * 16, 16), 16)] = jnp.zeros(
                (16,), jnp.float32)
            return carry

        lax.fori_loop(0, DSTRIPE // 16, zbody, 0)
        pltpu.sync_copy(zv, acc.at[pl.ds(sid * DSTRIPE, DSTRIPE)])
        plsc.subcore_barrier()

        def body(j, carry):
            base = pl.multiple_of((wid * KCH + j) * CHUNK, CHUNK)
            pltpu.sync_copy(col_hbm.at[pl.ds(base, CHUNK)], colv)
            pltpu.sync_copy(onesv, acc.at[colv], add=True)
            return carry

        lax.fori_loop(0, KCH, body, 0)
        plsc.subcore_barrier()
        off = pl.multiple_of(cid * NPAD + sid * DSTRIPE, 8)
        pltpu.sync_copy(acc.at[pl.ds(sid * DSTRIPE, DSTRIPE)],
                        out_hbm.at[pl.ds(off, DSTRIPE)])

    @functools.partial(
        pl.kernel,
        out_type=jax.ShapeDtypeStruct((2 * NPAD, D_out), jnp.float32),
        mesh=_mesh(),
        scratch_types=[
            pltpu.VMEM((CHUNK,), jnp.int32),
            pltpu.VMEM((CHUNK,), jnp.int32),
            pltpu.VMEM((CHUNK, D_out), jnp.float32),
            pltpu.SemaphoreType.DMA,
            pltpu.VMEM_SHARED((NPAD, D_out), jnp.float32),
        ],
    )
    def _agg(hp_hbm, row_hbm, col_hbm, zrows_hbm, out_hbm,
             rowv, colv, datav, sem, acc):
        cid = lax.axis_index("c")
        sid = lax.axis_index("s")
        wid = sid * NC + cid
        pltpu.sync_copy(zrows_hbm, acc.at[pl.ds(sid * RSTRIPE, RSTRIPE)])
        plsc.subcore_barrier()

        def body(j, carry):
            base = pl.multiple_of((wid * KCH + j) * CHUNK, CHUNK)
            pltpu.sync_copy(row_hbm.at[pl.ds(base, CHUNK)], rowv)
            pltpu.sync_copy(col_hbm.at[pl.ds(base, CHUNK)], colv)
            pltpu.async_copy(hp_hbm.at[rowv], datav, sem).wait()
            pltpu.sync_copy(datav, acc.at[colv], add=True)
            return carry

        lax.fori_loop(0, KCH, body, 0)
        plsc.subcore_barrier()
        pltpu.sync_copy(
            acc.at[pl.ds(sid * RSTRIPE, RSTRIPE)],
            out_hbm.at[pl.ds(cid * NPAD + sid * RSTRIPE, RSTRIPE)])

    MBLK = 1024
    GRID = NPAD // MBLK

    def _lin_body(xb, wb, d0b, d1b, hb):
        deg = 1.0 + d0b[...] + d1b[...]
        dis = lax.rsqrt(deg)
        h = jnp.dot(xb[...], wb[...], preferred_element_type=jnp.float32,
                    precision=lax.Precision.HIGHEST)
        hb[...] = h * dis

    _lin = pl.pallas_call(
        _lin_body,
        grid=(GRID,),
        in_specs=[
            pl.BlockSpec((MBLK, D_in), lambda i: (i, 0)),
            pl.BlockSpec((D_in, D_out), lambda i: (0, 0)),
            pl.BlockSpec((MBLK, 1), lambda i: (i, 0)),
            pl.BlockSpec((MBLK, 1), lambda i: (i, 0)),
        ],
        out_specs=pl.BlockSpec((MBLK, D_out), lambda i: (i, 0)),
        out_shape=jax.ShapeDtypeStruct((NPAD, D_out), jnp.float32),
    )

    def _fin_body(p0b, p1b, hb, d0b, d1b, bb, ob):
        deg = 1.0 + d0b[...] + d1b[...]
        dis = lax.rsqrt(deg)
        ob[...] = (p0b[...] + p1b[...] + hb[...]) * dis + bb[...]

    _fin = pl.pallas_call(
        _fin_body,
        grid=(GRID,),
        in_specs=[
            pl.BlockSpec((MBLK, D_out), lambda i: (i, 0)),
            pl.BlockSpec((MBLK, D_out), lambda i: (i, 0)),
            pl.BlockSpec((MBLK, D_out), lambda i: (i, 0)),
            pl.BlockSpec((MBLK, 1), lambda i: (i, 0)),
            pl.BlockSpec((MBLK, 1), lambda i: (i, 0)),
            pl.BlockSpec((1, D_out), lambda i: (0, 0)),
        ],
        out_specs=pl.BlockSpec((MBLK, D_out), lambda i: (i, 0)),
        out_shape=jax.ShapeDtypeStruct((NPAD, D_out), jnp.float32),
    )

    @jax.jit
    def run(x, edge_index, W, b):
        row = edge_index[0].astype(jnp.int32)
        col = edge_index[1].astype(jnp.int32)
        npad_e = E_PAD - E
        if npad_e:
            ar = jnp.arange(npad_e, dtype=jnp.int32)
            pad_row = (ar * 37) % N
            pad_col = N + ar % (NPAD - N)
            row = jnp.concatenate([row, pad_row])
            col = jnp.concatenate([col, pad_col])
        deg_flat = _deg(col)
        d0 = deg_flat[:NPAD].reshape(NPAD, 1)
        d1 = deg_flat[NPAD:].reshape(NPAD, 1)
        x_pad = jnp.zeros((NPAD, D_in), jnp.float32).at[:N].set(x)
        hp = _lin(x_pad, W, d0, d1)
        zrows = jnp.zeros((RSTRIPE, D_out), jnp.float32)
        agg_flat = _agg(hp, row, col, zrows)
        out_pad = _fin(agg_flat[:NPAD], agg_flat[NPAD:], hp, d0, d1,
                       b.reshape(1, D_out))
        return out_pad[:N]

    return run


def kernel(x, edge_index, W, b):
    N, D_in = x.shape
    D_out = W.shape[1]
    E = edge_index.shape[1]
    return _build(N, E, D_in, D_out)(x, edge_index, W, b)


# speedup vs baseline: 27.9663x; 1.3395x over previous
import functools

import jax
import jax.numpy as jnp
from jax import lax
from jax.experimental import pallas as pl
from jax.experimental.pallas import tpu as pltpu
from jax.experimental.pallas import tpu_sc as plsc

NC = 2
NS = 16
NW = NC * NS
CHUNK = 128


def _mesh():
    return plsc.VectorSubcoreMesh(
        core_axis_name="c", subcore_axis_name="s", num_cores=NC,
        num_subcores=NS)


@functools.lru_cache(maxsize=None)
def _build(N, E, D_in, D_out):
    NPAD = ((N + 1023) // 1024) * 1024
    KCH = -(-E // (NW * CHUNK))
    if KCH % 2:
        KCH += 1
    E_PAD = NW * CHUNK * KCH
    NCHUNK = E_PAD // CHUNK + 1
    DSTRIPE = NPAD // NS
    RSTRIPE = NPAD // NS

    @functools.partial(
        pl.kernel,
        out_type=jax.ShapeDtypeStruct((2 * NPAD,), jnp.float32),
        mesh=_mesh(),
        scratch_types=[
            pltpu.VMEM((CHUNK,), jnp.int32),
            pltpu.VMEM((CHUNK,), jnp.float32),
            pltpu.VMEM((DSTRIPE,), jnp.float32),
            pltpu.VMEM_SHARED((NPAD,), jnp.float32),
        ],
    )
    def _deg(col_hbm, out_hbm, colv, onesv, zv, acc):
        cid = lax.axis_index("c")
        sid = lax.axis_index("s")
        wid = sid * NC + cid
        for i in range(CHUNK // 16):
            onesv[pl.ds(i * 16, 16)] = jnp.ones((16,), jnp.float32)

        def zbody(i, carry):
            zv[pl.ds(pl.multiple_of(i * 16, 16), 16)] = jnp.zeros(
                (16,), jnp.float32)
            return carry

        lax.fori_loop(0, DSTRIPE // 16, zbody, 0)
        pltpu.sync_copy(zv, acc.at[pl.ds(sid * DSTRIPE, DSTRIPE)])
        plsc.subcore_barrier()

        def body(j, carry):
            pltpu.sync_copy(col_hbm.at[wid * KCH + j], colv)
            pltpu.sync_copy(onesv, acc.at[colv], add=True)
            return carry

        lax.fori_loop(0, KCH, body, 0)
        plsc.subcore_barrier()
        off = pl.multiple_of(cid * NPAD + sid * DSTRIPE, 8)
        pltpu.sync_copy(acc.at[pl.ds(sid * DSTRIPE, DSTRIPE)],
                        out_hbm.at[pl.ds(off, DSTRIPE)])

    @functools.partial(
        pl.kernel,
        out_type=jax.ShapeDtypeStruct((2 * NPAD, D_out), jnp.float32),
        mesh=_mesh(),
        scratch_types=[
            pltpu.VMEM((2, CHUNK), jnp.int32),
            pltpu.VMEM((2, CHUNK), jnp.int32),
            pltpu.VMEM((2, CHUNK, D_out), jnp.float32),
            pltpu.SemaphoreType.DMA,
            pltpu.SemaphoreType.DMA,
            pltpu.VMEM_SHARED((NPAD, D_out), jnp.float32),
        ],
    )
    def _agg(hp_hbm, row_hbm, col_hbm, zrows_hbm, out_hbm,
             rowv, colv, datav, sem0, sem1, acc):
        cid = lax.axis_index("c")
        sid = lax.axis_index("s")
        wid = sid * NC + cid
        sems = (sem0, sem1)
        pltpu.sync_copy(zrows_hbm, acc.at[pl.ds(sid * RSTRIPE, RSTRIPE)])
        plsc.subcore_barrier()

        pltpu.sync_copy(row_hbm.at[wid * KCH], rowv.at[0])
        pltpu.sync_copy(col_hbm.at[wid * KCH], colv.at[0])
        pltpu.async_copy(hp_hbm.at[rowv.at[0]], datav.at[0], sem0)

        def body(jj, carry):
            for b in range(2):
                j = jj * 2 + b
                nb = 1 - b
                pltpu.sync_copy(row_hbm.at[wid * KCH + j + 1], rowv.at[nb])
                pltpu.sync_copy(col_hbm.at[wid * KCH + j + 1], colv.at[nb])
                pltpu.async_copy(hp_hbm.at[rowv.at[nb]], datav.at[nb],
                                 sems[nb])
                pltpu.make_async_copy(hp_hbm.at[rowv.at[b]], datav.at[b],
                                      sems[b]).wait()
                pltpu.sync_copy(datav.at[b], acc.at[colv.at[b]], add=True)
            return carry

        lax.fori_loop(0, KCH // 2, body, 0)
        pltpu.make_async_copy(hp_hbm.at[rowv.at[0]], datav.at[0],
                              sem0).wait()
        plsc.subcore_barrier()
        pltpu.sync_copy(
            acc.at[pl.ds(sid * RSTRIPE, RSTRIPE)],
            out_hbm.at[pl.ds(cid * NPAD + sid * RSTRIPE, RSTRIPE)])

    MBLK = 1024
    GRID = NPAD // MBLK

    def _lin_body(xb, wb, d0b, d1b, hb):
        deg = 1.0 + d0b[...] + d1b[...]
        dis = lax.rsqrt(deg)
        h = jnp.dot(xb[...], wb[...], preferred_element_type=jnp.float32,
                    precision=lax.Precision.HIGHEST)
        hb[...] = h * dis

    _lin = pl.pallas_call(
        _lin_body,
        grid=(GRID,),
        in_specs=[
            pl.BlockSpec((MBLK, D_in), lambda i: (i, 0)),
            pl.BlockSpec((D_in, D_out), lambda i: (0, 0)),
            pl.BlockSpec((MBLK, 1), lambda i: (i, 0)),
            pl.BlockSpec((MBLK, 1), lambda i: (i, 0)),
        ],
        out_specs=pl.BlockSpec((MBLK, D_out), lambda i: (i, 0)),
        out_shape=jax.ShapeDtypeStruct((NPAD, D_out), jnp.float32),
    )

    def _fin_body(p0b, p1b, hb, d0b, d1b, bb, ob):
        deg = 1.0 + d0b[...] + d1b[...]
        dis = lax.rsqrt(deg)
        ob[...] = (p0b[...] + p1b[...] + hb[...]) * dis + bb[...]

    _fin = pl.pallas_call(
        _fin_body,
        grid=(GRID,),
        in_specs=[
            pl.BlockSpec((MBLK, D_out), lambda i: (i, 0)),
            pl.BlockSpec((MBLK, D_out), lambda i: (i, 0)),
            pl.BlockSpec((MBLK, D_out), lambda i: (i, 0)),
            pl.BlockSpec((MBLK, 1), lambda i: (i, 0)),
            pl.BlockSpec((MBLK, 1), lambda i: (i, 0)),
            pl.BlockSpec((1, D_out), lambda i: (0, 0)),
        ],
        out_specs=pl.BlockSpec((MBLK, D_out), lambda i: (i, 0)),
        out_shape=jax.ShapeDtypeStruct((NPAD, D_out), jnp.float32),
    )

    @jax.jit
    def run(x, edge_index, W, b):
        row = edge_index[0].astype(jnp.int32)
        col = edge_index[1].astype(jnp.int32)
        npad_e = NCHUNK * CHUNK - E
        ar = jnp.arange(npad_e, dtype=jnp.int32)
        pad_row = (ar * 37) % N
        pad_col = N + ar % (NPAD - N)
        row = jnp.concatenate([row, pad_row]).reshape(NCHUNK, CHUNK)
        col = jnp.concatenate([col, pad_col]).reshape(NCHUNK, CHUNK)
        deg_flat = _deg(col)
        d0 = deg_flat[:NPAD].reshape(NPAD, 1)
        d1 = deg_flat[NPAD:].reshape(NPAD, 1)
        x_pad = jnp.zeros((NPAD, D_in), jnp.float32).at[:N].set(x)
        hp = _lin(x_pad, W, d0, d1)
        zrows = jnp.zeros((RSTRIPE, D_out), jnp.float32)
        agg_flat = _agg(hp, row, col, zrows)
        out_pad = _fin(agg_flat[:NPAD], agg_flat[NPAD:], hp, d0, d1,
                       b.reshape(1, D_out))
        return out_pad[:N]

    return run


def kernel(x, edge_index, W, b):
    N, D_in = x.shape
    D_out = W.shape[1]
    E = edge_index.shape[1]
    return _build(N, E, D_in, D_out)(x, edge_index, W, b)


# speedup vs baseline: 32.3495x; 1.1567x over previous
import functools

import jax
import jax.numpy as jnp
from jax import lax
from jax.experimental import pallas as pl
from jax.experimental.pallas import tpu as pltpu
from jax.experimental.pallas import tpu_sc as plsc

NC = 2
NS = 16
NW = NC * NS
CHUNK = 128


def _mesh():
    return plsc.VectorSubcoreMesh(
        core_axis_name="c", subcore_axis_name="s", num_cores=NC,
        num_subcores=NS)


@functools.lru_cache(maxsize=None)
def _build(N, E, D_in, D_out):
    NPAD = ((N + 1023) // 1024) * 1024
    KCH = -(-E // (NW * CHUNK))
    KCH = ((KCH + 15) // 16) * 16
    E_PAD = NW * CHUNK * KCH
    SUP = 8
    NCHUNK = E_PAD // CHUNK + SUP
    DSTRIPE = NPAD // NS
    RSTRIPE = NPAD // NS

    @functools.partial(
        pl.kernel,
        out_type=jax.ShapeDtypeStruct((2 * NPAD,), jnp.float32),
        mesh=_mesh(),
        scratch_types=[
            pltpu.VMEM((2, SUP, CHUNK), jnp.int32),
            pltpu.VMEM((CHUNK,), jnp.float32),
            pltpu.VMEM((DSTRIPE,), jnp.float32),
            pltpu.SemaphoreType.DMA,
            pltpu.SemaphoreType.DMA,
            pltpu.VMEM_SHARED((NPAD,), jnp.float32),
        ],
    )
    def _deg(col_hbm, out_hbm, colv, onesv, zv, sem0, sem1, acc):
        cid = lax.axis_index("c")
        sid = lax.axis_index("s")
        wid = sid * NC + cid
        sems = (sem0, sem1)
        for i in range(CHUNK // 16):
            onesv[pl.ds(i * 16, 16)] = jnp.ones((16,), jnp.float32)

        def zbody(i, carry):
            zv[pl.ds(pl.multiple_of(i * 16, 16), 16)] = jnp.zeros(
                (16,), jnp.float32)
            return carry

        lax.fori_loop(0, DSTRIPE // 16, zbody, 0)
        pltpu.sync_copy(zv, acc.at[pl.ds(sid * DSTRIPE, DSTRIPE)])
        plsc.subcore_barrier()

        nsup = KCH // SUP
        base = wid * nsup

        def fire(b):
            for s in range(SUP):
                pltpu.async_copy(onesv, acc.at[colv.at[b, s]], sems[b],
                                 add=True)

        def drain(b):
            for s in range(SUP):
                pltpu.make_async_copy(onesv, acc.at[colv.at[b, s]],
                                      sems[b]).wait()

        pltpu.sync_copy(col_hbm.at[pl.ds(base * SUP, SUP)], colv.at[0])
        fire(0)

        def body(jj, carry):
            for b in range(2):
                nb = 1 - b
                sc = jj * 2 + b
                pltpu.sync_copy(
                    col_hbm.at[pl.ds((base + sc + 1) * SUP, SUP)],
                    colv.at[nb])
                drain(b)
                fire(nb)
            return carry

        lax.fori_loop(0, nsup // 2, body, 0)
        drain(0)
        plsc.subcore_barrier()
        off = pl.multiple_of(cid * NPAD + sid * DSTRIPE, 8)
        pltpu.sync_copy(acc.at[pl.ds(sid * DSTRIPE, DSTRIPE)],
                        out_hbm.at[pl.ds(off, DSTRIPE)])

    @functools.partial(
        pl.kernel,
        out_type=jax.ShapeDtypeStruct((2 * NPAD, D_out), jnp.float32),
        mesh=_mesh(),
        scratch_types=[
            pltpu.VMEM((2, CHUNK), jnp.int32),
            pltpu.VMEM((2, CHUNK), jnp.int32),
            pltpu.VMEM((2, CHUNK, D_out), jnp.float32),
            pltpu.SemaphoreType.DMA,
            pltpu.SemaphoreType.DMA,
            pltpu.VMEM_SHARED((NPAD, D_out), jnp.float32),
        ],
    )
    def _agg(hp_hbm, row_hbm, col_hbm, zrows_hbm, out_hbm,
             rowv, colv, datav, sem0, sem1, acc):
        cid = lax.axis_index("c")
        sid = lax.axis_index("s")
        wid = sid * NC + cid
        sems = (sem0, sem1)
        pltpu.sync_copy(zrows_hbm, acc.at[pl.ds(sid * RSTRIPE, RSTRIPE)])
        plsc.subcore_barrier()

        pltpu.sync_copy(row_hbm.at[wid * KCH], rowv.at[0])
        pltpu.sync_copy(col_hbm.at[wid * KCH], colv.at[0])
        pltpu.async_copy(hp_hbm.at[rowv.at[0]], datav.at[0], sem0)

        def body(jj, carry):
            for b in range(2):
                j = jj * 2 + b
                nb = 1 - b
                pltpu.sync_copy(row_hbm.at[wid * KCH + j + 1], rowv.at[nb])
                pltpu.sync_copy(col_hbm.at[wid * KCH + j + 1], colv.at[nb])
                pltpu.async_copy(hp_hbm.at[rowv.at[nb]], datav.at[nb],
                                 sems[nb])
                pltpu.make_async_copy(hp_hbm.at[rowv.at[b]], datav.at[b],
                                      sems[b]).wait()
                pltpu.sync_copy(datav.at[b], acc.at[colv.at[b]], add=True)
            return carry

        lax.fori_loop(0, KCH // 2, body, 0)
        pltpu.make_async_copy(hp_hbm.at[rowv.at[0]], datav.at[0],
                              sem0).wait()
        plsc.subcore_barrier()
        pltpu.sync_copy(
            acc.at[pl.ds(sid * RSTRIPE, RSTRIPE)],
            out_hbm.at[pl.ds(cid * NPAD + sid * RSTRIPE, RSTRIPE)])

    MBLK = 1024
    GRID = NPAD // MBLK

    def _lin_body(xb, wb, d0b, d1b, hb):
        deg = 1.0 + d0b[...] + d1b[...]
        dis = lax.rsqrt(deg)
        h = jnp.dot(xb[...], wb[...], preferred_element_type=jnp.float32,
                    precision=lax.Precision.HIGHEST)
        hb[...] = h * dis

    _lin = pl.pallas_call(
        _lin_body,
        grid=(GRID,),
        in_specs=[
            pl.BlockSpec((MBLK, D_in), lambda i: (i, 0)),
            pl.BlockSpec((D_in, D_out), lambda i: (0, 0)),
            pl.BlockSpec((MBLK, 1), lambda i: (i, 0)),
            pl.BlockSpec((MBLK, 1), lambda i: (i, 0)),
        ],
        out_specs=pl.BlockSpec((MBLK, D_out), lambda i: (i, 0)),
        out_shape=jax.ShapeDtypeStruct((NPAD, D_out), jnp.float32),
    )

    def _fin_body(p0b, p1b, hb, d0b, d1b, bb, ob):
        deg = 1.0 + d0b[...] + d1b[...]
        dis = lax.rsqrt(deg)
        ob[...] = (p0b[...] + p1b[...] + hb[...]) * dis + bb[...]

    _fin = pl.pallas_call(
        _fin_body,
        grid=(GRID,),
        in_specs=[
            pl.BlockSpec((MBLK, D_out), lambda i: (i, 0)),
            pl.BlockSpec((MBLK, D_out), lambda i: (i, 0)),
            pl.BlockSpec((MBLK, D_out), lambda i: (i, 0)),
            pl.BlockSpec((MBLK, 1), lambda i: (i, 0)),
            pl.BlockSpec((MBLK, 1), lambda i: (i, 0)),
            pl.BlockSpec((1, D_out), lambda i: (0, 0)),
        ],
        out_specs=pl.BlockSpec((MBLK, D_out), lambda i: (i, 0)),
        out_shape=jax.ShapeDtypeStruct((NPAD, D_out), jnp.float32),
    )

    @jax.jit
    def run(x, edge_index, W, b):
        row = edge_index[0].astype(jnp.int32)
        col = edge_index[1].astype(jnp.int32)
        npad_e = NCHUNK * CHUNK - E
        ar = jnp.arange(npad_e, dtype=jnp.int32)
        pad_row = (ar * 37) % N
        pad_col = N + ar % (NPAD - N)
        row = jnp.concatenate([row, pad_row]).reshape(NCHUNK, CHUNK)
        col = jnp.concatenate([col, pad_col]).reshape(NCHUNK, CHUNK)
        deg_flat = _deg(col)
        d0 = deg_flat[:NPAD].reshape(NPAD, 1)
        d1 = deg_flat[NPAD:].reshape(NPAD, 1)
        x_pad = jnp.zeros((NPAD, D_in), jnp.float32).at[:N].set(x)
        hp = _lin(x_pad, W, d0, d1)
        zrows = jnp.zeros((RSTRIPE, D_out), jnp.float32)
        agg_flat = _agg(hp, row, col, zrows)
        out_pad = _fin(agg_flat[:NPAD], agg_flat[NPAD:], hp, d0, d1,
                       b.reshape(1, D_out))
        return out_pad[:N]

    return run


def kernel(x, edge_index, W, b):
    N, D_in = x.shape
    D_out = W.shape[1]
    E = edge_index.shape[1]
    return _build(N, E, D_in, D_out)(x, edge_index, W, b)


# speedup vs baseline: 32.4436x; 1.0029x over previous
import functools

import jax
import jax.numpy as jnp
from jax import lax
from jax.experimental import pallas as pl
from jax.experimental.pallas import tpu as pltpu
from jax.experimental.pallas import tpu_sc as plsc

NC = 2
NS = 16
NW = NC * NS
CHUNK = 128


def _mesh():
    return plsc.VectorSubcoreMesh(
        core_axis_name="c", subcore_axis_name="s", num_cores=NC,
        num_subcores=NS)


@functools.lru_cache(maxsize=None)
def _build(N, E, D_in, D_out):
    NPAD = ((N + 1023) // 1024) * 1024
    KCH = -(-E // (NW * CHUNK))
    KCH = ((KCH + 15) // 16) * 16
    E_PAD = NW * CHUNK * KCH
    SUP = 8
    NCHUNK = E_PAD // CHUNK + SUP
    DSTRIPE = NPAD // NS
    RSTRIPE = NPAD // NS

    @functools.partial(
        pl.kernel,
        out_type=jax.ShapeDtypeStruct((2 * NPAD,), jnp.float32),
        mesh=_mesh(),
        scratch_types=[
            pltpu.VMEM((2, SUP, CHUNK), jnp.int32),
            pltpu.VMEM((CHUNK,), jnp.float32),
            pltpu.VMEM((DSTRIPE,), jnp.float32),
            pltpu.SemaphoreType.DMA,
            pltpu.SemaphoreType.DMA,
            pltpu.VMEM_SHARED((NPAD,), jnp.float32),
        ],
    )
    def _deg(col_hbm, out_hbm, colv, onesv, zv, sem0, sem1, acc):
        cid = lax.axis_index("c")
        sid = lax.axis_index("s")
        wid = sid * NC + cid
        sems = (sem0, sem1)
        for i in range(CHUNK // 16):
            onesv[pl.ds(i * 16, 16)] = jnp.ones((16,), jnp.float32)

        def zbody(i, carry):
            zv[pl.ds(pl.multiple_of(i * 16, 16), 16)] = jnp.zeros(
                (16,), jnp.float32)
            return carry

        lax.fori_loop(0, DSTRIPE // 16, zbody, 0)
        pltpu.sync_copy(zv, acc.at[pl.ds(sid * DSTRIPE, DSTRIPE)])
        plsc.subcore_barrier()

        nsup = KCH // SUP
        base = wid * nsup

        def fire(b):
            for s in range(SUP):
                pltpu.async_copy(onesv, acc.at[colv.at[b, s]], sems[b],
                                 add=True)

        def drain(b):
            for s in range(SUP):
                pltpu.make_async_copy(onesv, acc.at[colv.at[b, s]],
                                      sems[b]).wait()

        pltpu.sync_copy(col_hbm.at[pl.ds(base * SUP, SUP)], colv.at[0])
        fire(0)

        def body(jj, carry):
            for b in range(2):
                nb = 1 - b
                sc = jj * 2 + b
                pltpu.sync_copy(
                    col_hbm.at[pl.ds((base + sc + 1) * SUP, SUP)],
                    colv.at[nb])
                drain(b)
                if b == 0:
                    fire(nb)
                else:
                    @pl.when(jj < nsup // 2 - 1)
                    def _():
                        fire(nb)
            return carry

        lax.fori_loop(0, nsup // 2, body, 0)
        plsc.subcore_barrier()
        off = pl.multiple_of(cid * NPAD + sid * DSTRIPE, 8)
        pltpu.sync_copy(acc.at[pl.ds(sid * DSTRIPE, DSTRIPE)],
                        out_hbm.at[pl.ds(off, DSTRIPE)])

    @functools.partial(
        pl.kernel,
        out_type=jax.ShapeDtypeStruct((2 * NPAD, D_out), jnp.float32),
        mesh=_mesh(),
        scratch_types=[
            pltpu.VMEM((2, CHUNK), jnp.int32),
            pltpu.VMEM((2, CHUNK), jnp.int32),
            pltpu.VMEM((2, CHUNK, D_out), jnp.float32),
            pltpu.SemaphoreType.DMA,
            pltpu.SemaphoreType.DMA,
            pltpu.VMEM_SHARED((NPAD, D_out), jnp.float32),
        ],
    )
    def _agg(hp_hbm, row_hbm, col_hbm, zrows_hbm, out_hbm,
             rowv, colv, datav, sem0, sem1, acc):
        cid = lax.axis_index("c")
        sid = lax.axis_index("s")
        wid = sid * NC + cid
        sems = (sem0, sem1)
        pltpu.sync_copy(zrows_hbm, acc.at[pl.ds(sid * RSTRIPE, RSTRIPE)])
        plsc.subcore_barrier()

        pltpu.sync_copy(row_hbm.at[wid * KCH], rowv.at[0])
        pltpu.sync_copy(col_hbm.at[wid * KCH], colv.at[0])
        pltpu.async_copy(hp_hbm.at[rowv.at[0]], datav.at[0], sem0)

        def body(jj, carry):
            for b in range(2):
                j = jj * 2 + b
                nb = 1 - b
                pltpu.sync_copy(row_hbm.at[wid * KCH + j + 1], rowv.at[nb])
                pltpu.sync_copy(col_hbm.at[wid * KCH + j + 1], colv.at[nb])
                pltpu.async_copy(hp_hbm.at[rowv.at[nb]], datav.at[nb],
                                 sems[nb])
                pltpu.make_async_copy(hp_hbm.at[rowv.at[b]], datav.at[b],
                                      sems[b]).wait()
                pltpu.sync_copy(datav.at[b], acc.at[colv.at[b]], add=True)
            return carry

        lax.fori_loop(0, KCH // 2, body, 0)
        pltpu.make_async_copy(hp_hbm.at[rowv.at[0]], datav.at[0],
                              sem0).wait()
        plsc.subcore_barrier()
        pltpu.sync_copy(
            acc.at[pl.ds(sid * RSTRIPE, RSTRIPE)],
            out_hbm.at[pl.ds(cid * NPAD + sid * RSTRIPE, RSTRIPE)])

    MBLK = 1024
    GRID = NPAD // MBLK

    def _lin_body(xb, wb, d0b, d1b, hb):
        deg = 1.0 + d0b[...] + d1b[...]
        dis = lax.rsqrt(deg)
        h = jnp.dot(xb[...], wb[...], preferred_element_type=jnp.float32,
                    precision=lax.Precision.HIGHEST)
        hb[...] = h * dis

    _lin = pl.pallas_call(
        _lin_body,
        grid=(GRID,),
        in_specs=[
            pl.BlockSpec((MBLK, D_in), lambda i: (i, 0)),
            pl.BlockSpec((D_in, D_out), lambda i: (0, 0)),
            pl.BlockSpec((MBLK, 1), lambda i: (i, 0)),
            pl.BlockSpec((MBLK, 1), lambda i: (i, 0)),
        ],
        out_specs=pl.BlockSpec((MBLK, D_out), lambda i: (i, 0)),
        out_shape=jax.ShapeDtypeStruct((NPAD, D_out), jnp.float32),
    )

    def _fin_body(p0b, p1b, hb, d0b, d1b, bb, ob):
        deg = 1.0 + d0b[...] + d1b[...]
        dis = lax.rsqrt(deg)
        ob[...] = (p0b[...] + p1b[...] + hb[...]) * dis + bb[...]

    _fin = pl.pallas_call(
        _fin_body,
        grid=(GRID,),
        in_specs=[
            pl.BlockSpec((MBLK, D_out), lambda i: (i, 0)),
            pl.BlockSpec((MBLK, D_out), lambda i: (i, 0)),
            pl.BlockSpec((MBLK, D_out), lambda i: (i, 0)),
            pl.BlockSpec((MBLK, 1), lambda i: (i, 0)),
            pl.BlockSpec((MBLK, 1), lambda i: (i, 0)),
            pl.BlockSpec((1, D_out), lambda i: (0, 0)),
        ],
        out_specs=pl.BlockSpec((MBLK, D_out), lambda i: (i, 0)),
        out_shape=jax.ShapeDtypeStruct((NPAD, D_out), jnp.float32),
    )

    @jax.jit
    def run(x, edge_index, W, b):
        row = edge_index[0].astype(jnp.int32)
        col = edge_index[1].astype(jnp.int32)
        npad_e = NCHUNK * CHUNK - E
        ar = jnp.arange(npad_e, dtype=jnp.int32)
        pad_row = (ar * 37) % N
        pad_col = N + ar % (NPAD - N)
        row = jnp.concatenate([row, pad_row]).reshape(NCHUNK, CHUNK)
        col = jnp.concatenate([col, pad_col]).reshape(NCHUNK, CHUNK)
        deg_flat = _deg(col)
        d0 = deg_flat[:NPAD].reshape(NPAD, 1)
        d1 = deg_flat[NPAD:].reshape(NPAD, 1)
        x_pad = jnp.zeros((NPAD, D_in), jnp.float32).at[:N].set(x)
        hp = _lin(x_pad, W, d0, d1)
        zrows = jnp.zeros((RSTRIPE, D_out), jnp.float32)
        agg_flat = _agg(hp, row, col, zrows)
        out_pad = _fin(agg_flat[:NPAD], agg_flat[NPAD:], hp, d0, d1,
                       b.reshape(1, D_out))
        return out_pad[:N]

    return run


def kernel(x, edge_index, W, b):
    N, D_in = x.shape
    D_out = W.shape[1]
    E = edge_index.shape[1]
    return _build(N, E, D_in, D_out)(x, edge_index, W, b)


# speedup vs baseline: 33.7908x; 1.0415x over previous
import functools

import jax
import jax.numpy as jnp
from jax import lax
from jax.experimental import pallas as pl
from jax.experimental.pallas import tpu as pltpu
from jax.experimental.pallas import tpu_sc as plsc

NC = 2
NS = 16
NW = NC * NS
CHUNK = 128


def _mesh():
    return plsc.VectorSubcoreMesh(
        core_axis_name="c", subcore_axis_name="s", num_cores=NC,
        num_subcores=NS)


@functools.lru_cache(maxsize=None)
def _build(N, E, D_in, D_out):
    NPAD = ((N + 1023) // 1024) * 1024
    KCH = -(-E // (NW * CHUNK))
    KCH = ((KCH + 15) // 16) * 16
    E_PAD = NW * CHUNK * KCH
    SUP = 8
    NCHUNK = E_PAD // CHUNK + SUP
    DSTRIPE = NPAD // NS
    RSTRIPE = NPAD // NS

    @functools.partial(
        pl.kernel,
        out_type=jax.ShapeDtypeStruct((2 * NPAD,), jnp.float32),
        mesh=_mesh(),
        scratch_types=[
            pltpu.VMEM((2, SUP, CHUNK), jnp.int32),
            pltpu.VMEM((CHUNK,), jnp.float32),
            pltpu.VMEM((DSTRIPE,), jnp.float32),
            pltpu.SemaphoreType.DMA,
            pltpu.SemaphoreType.DMA,
            pltpu.VMEM_SHARED((NPAD,), jnp.float32),
        ],
    )
    def _deg(col_hbm, out_hbm, colv, onesv, zv, sem0, sem1, acc):
        cid = lax.axis_index("c")
        sid = lax.axis_index("s")
        wid = sid * NC + cid
        sems = (sem0, sem1)
        for i in range(CHUNK // 16):
            onesv[pl.ds(i * 16, 16)] = jnp.ones((16,), jnp.float32)

        def zbody(i, carry):
            zv[pl.ds(pl.multiple_of(i * 16, 16), 16)] = jnp.zeros(
                (16,), jnp.float32)
            return carry

        lax.fori_loop(0, DSTRIPE // 16, zbody, 0)
        pltpu.sync_copy(zv, acc.at[pl.ds(sid * DSTRIPE, DSTRIPE)])
        plsc.subcore_barrier()

        nsup = KCH // SUP
        base = wid * nsup

        def fire(b):
            for s in range(SUP):
                pltpu.async_copy(onesv, acc.at[colv.at[b, s]], sems[b],
                                 add=True)

        def drain(b):
            for s in range(SUP):
                pltpu.make_async_copy(onesv, acc.at[colv.at[b, s]],
                                      sems[b]).wait()

        pltpu.sync_copy(col_hbm.at[pl.ds(base * SUP, SUP)], colv.at[0])
        fire(0)

        def body(jj, carry):
            for b in range(2):
                nb = 1 - b
                sc = jj * 2 + b
                pltpu.sync_copy(
                    col_hbm.at[pl.ds((base + sc + 1) * SUP, SUP)],
                    colv.at[nb])
                drain(b)
                if b == 0:
                    fire(nb)
                else:
                    @pl.when(jj < nsup // 2 - 1)
                    def _():
                        fire(nb)
            return carry

        lax.fori_loop(0, nsup // 2, body, 0)
        plsc.subcore_barrier()
        off = pl.multiple_of(cid * NPAD + sid * DSTRIPE, 8)
        pltpu.sync_copy(acc.at[pl.ds(sid * DSTRIPE, DSTRIPE)],
                        out_hbm.at[pl.ds(off, DSTRIPE)])

    @functools.partial(
        pl.kernel,
        out_type=jax.ShapeDtypeStruct((2 * NPAD, D_out), jnp.float32),
        mesh=_mesh(),
        scratch_types=[
            pltpu.VMEM((2, SUP, CHUNK), jnp.int32),
            pltpu.VMEM((2, SUP, CHUNK), jnp.int32),
            pltpu.VMEM((2, CHUNK, D_out), jnp.float32),
            [pltpu.SemaphoreType.DMA] * 2,
            [pltpu.SemaphoreType.DMA] * 2,
            pltpu.VMEM_SHARED((NPAD, D_out), jnp.float32),
        ],
    )
    def _agg(hp_hbm, row_hbm, col_hbm, zrows_hbm, out_hbm,
             rowv, colv, datav, gsems, ssems, acc):
        cid = lax.axis_index("c")
        sid = lax.axis_index("s")
        wid = sid * NC + cid
        pltpu.sync_copy(zrows_hbm, acc.at[pl.ds(sid * RSTRIPE, RSTRIPE)])
        plsc.subcore_barrier()

        nsup = KCH // SUP
        base = wid * nsup

        def ld_sup(s, ib):
            pltpu.sync_copy(row_hbm.at[pl.ds((base + s) * SUP, SUP)],
                            rowv.at[ib])
            pltpu.sync_copy(col_hbm.at[pl.ds((base + s) * SUP, SUP)],
                            colv.at[ib])

        def g_fire(ib, c, b):
            pltpu.async_copy(hp_hbm.at[rowv.at[ib, c]], datav.at[b],
                             gsems[b])

        def g_wait(ib, c, b):
            pltpu.make_async_copy(hp_hbm.at[rowv.at[ib, c]], datav.at[b],
                                  gsems[b]).wait()

        def s_fire(ib, c, b):
            pltpu.async_copy(datav.at[b], acc.at[colv.at[ib, c]], ssems[b],
                             add=True)

        def s_wait(ib, c, b):
            pltpu.make_async_copy(datav.at[b], acc.at[colv.at[ib, c]],
                                  ssems[b]).wait()

        ld_sup(0, 0)
        g_fire(0, 0, 0)

        def body(tt, carry):
            for sb in range(2):
                t = tt * 2 + sb
                for c in range(SUP):
                    j = t * SUP + c
                    b = c % 2
                    nb = 1 - b
                    g_wait(sb, c, b)
                    s_fire(sb, c, b)
                    cp = (c + SUP - 1) % SUP
                    ibp = sb if c >= 1 else 1 - sb
                    if sb == 0 and c == 0:
                        @pl.when(tt > 0)
                        def _():
                            s_wait(ibp, cp, nb)
                    else:
                        s_wait(ibp, cp, nb)
                    if c == 0:
                        ld_sup(t + 1, 1 - sb)
                    if c < SUP - 1:
                        g_fire(sb, c + 1, nb)
                    else:
                        g_fire(1 - sb, 0, nb)
            return carry

        lax.fori_loop(0, nsup // 2, body, 0)
        s_wait(1, SUP - 1, 1)
        g_wait(0, 0, 0)
        plsc.subcore_barrier()
        pltpu.sync_copy(
            acc.at[pl.ds(sid * RSTRIPE, RSTRIPE)],
            out_hbm.at[pl.ds(cid * NPAD + sid * RSTRIPE, RSTRIPE)])

    MBLK = 1024
    GRID = NPAD // MBLK

    def _lin_body(xb, wb, d0b, d1b, hb):
        deg = 1.0 + d0b[...] + d1b[...]
        dis = lax.rsqrt(deg)
        h = jnp.dot(xb[...], wb[...], preferred_element_type=jnp.float32,
                    precision=lax.Precision.HIGHEST)
        hb[...] = h * dis

    _lin = pl.pallas_call(
        _lin_body,
        grid=(GRID,),
        in_specs=[
            pl.BlockSpec((MBLK, D_in), lambda i: (i, 0)),
            pl.BlockSpec((D_in, D_out), lambda i: (0, 0)),
            pl.BlockSpec((MBLK, 1), lambda i: (i, 0)),
            pl.BlockSpec((MBLK, 1), lambda i: (i, 0)),
        ],
        out_specs=pl.BlockSpec((MBLK, D_out), lambda i: (i, 0)),
        out_shape=jax.ShapeDtypeStruct((NPAD, D_out), jnp.float32),
    )

    def _fin_body(p0b, p1b, hb, d0b, d1b, bb, ob):
        deg = 1.0 + d0b[...] + d1b[...]
        dis = lax.rsqrt(deg)
        ob[...] = (p0b[...] + p1b[...] + hb[...]) * dis + bb[...]

    _fin = pl.pallas_call(
        _fin_body,
        grid=(GRID,),
        in_specs=[
            pl.BlockSpec((MBLK, D_out), lambda i: (i, 0)),
            pl.BlockSpec((MBLK, D_out), lambda i: (i, 0)),
            pl.BlockSpec((MBLK, D_out), lambda i: (i, 0)),
            pl.BlockSpec((MBLK, 1), lambda i: (i, 0)),
            pl.BlockSpec((MBLK, 1), lambda i: (i, 0)),
            pl.BlockSpec((1, D_out), lambda i: (0, 0)),
        ],
        out_specs=pl.BlockSpec((MBLK, D_out), lambda i: (i, 0)),
        out_shape=jax.ShapeDtypeStruct((NPAD, D_out), jnp.float32),
    )

    @jax.jit
    def run(x, edge_index, W, b):
        row = edge_index[0].astype(jnp.int32)
        col = edge_index[1].astype(jnp.int32)
        npad_e = NCHUNK * CHUNK - E
        ar = jnp.arange(npad_e, dtype=jnp.int32)
        pad_row = (ar * 37) % N
        pad_col = N + ar % (NPAD - N)
        row = jnp.concatenate([row, pad_row]).reshape(NCHUNK, CHUNK)
        col = jnp.concatenate([col, pad_col]).reshape(NCHUNK, CHUNK)
        deg_flat = _deg(col)
        d0 = deg_flat[:NPAD].reshape(NPAD, 1)
        d1 = deg_flat[NPAD:].reshape(NPAD, 1)
        x_pad = jnp.zeros((NPAD, D_in), jnp.float32).at[:N].set(x)
        hp = _lin(x_pad, W, d0, d1)
        zrows = jnp.zeros((RSTRIPE, D_out), jnp.float32)
        agg_flat = _agg(hp, row, col, zrows)
        out_pad = _fin(agg_flat[:NPAD], agg_flat[NPAD:], hp, d0, d1,
                       b.reshape(1, D_out))
        return out_pad[:N]

    return run


def kernel(x, edge_index, W, b):
    N, D_in = x.shape
    D_out = W.shape[1]
    E = edge_index.shape[1]
    return _build(N, E, D_in, D_out)(x, edge_index, W, b)


# speedup vs baseline: 37.7073x; 1.1159x over previous
import functools

import jax
import jax.numpy as jnp
from jax import lax
from jax.experimental import pallas as pl
from jax.experimental.pallas import tpu as pltpu
from jax.experimental.pallas import tpu_sc as plsc

NC = 2
NS = 16
NW = NC * NS
CHUNK = 128


def _mesh():
    return plsc.VectorSubcoreMesh(
        core_axis_name="c", subcore_axis_name="s", num_cores=NC,
        num_subcores=NS)


@functools.lru_cache(maxsize=None)
def _build(N, E, D_in, D_out):
    NPAD = ((N + 1023) // 1024) * 1024
    KCH = -(-E // (NW * CHUNK))
    KCH = ((KCH + 15) // 16) * 16
    E_PAD = NW * CHUNK * KCH
    SUP = 8
    NCHUNK = E_PAD // CHUNK + SUP
    DSTRIPE = NPAD // NS
    RSTRIPE = NPAD // NS

    @functools.partial(
        pl.kernel,
        out_type=jax.ShapeDtypeStruct((2 * NPAD,), jnp.float32),
        mesh=_mesh(),
        scratch_types=[
            pltpu.VMEM((2, SUP, CHUNK), jnp.int32),
            pltpu.VMEM((CHUNK,), jnp.float32),
            pltpu.VMEM((DSTRIPE,), jnp.float32),
            pltpu.SemaphoreType.DMA,
            pltpu.SemaphoreType.DMA,
            pltpu.VMEM_SHARED((NPAD,), jnp.float32),
        ],
    )
    def _deg(ei_hbm, out_hbm, colv, onesv, zv, sem0, sem1, acc):
        cid = lax.axis_index("c")
        sid = lax.axis_index("s")
        wid = sid * NC + cid
        sems = (sem0, sem1)
        for i in range(CHUNK // 16):
            onesv[pl.ds(i * 16, 16)] = jnp.ones((16,), jnp.float32)

        def zbody(i, carry):
            zv[pl.ds(pl.multiple_of(i * 16, 16), 16)] = jnp.zeros(
                (16,), jnp.float32)
            return carry

        lax.fori_loop(0, DSTRIPE // 16, zbody, 0)
        pltpu.sync_copy(zv, acc.at[pl.ds(sid * DSTRIPE, DSTRIPE)])
        plsc.subcore_barrier()

        nsup = KCH // SUP
        base = wid * nsup

        def fire(b):
            for s in range(SUP):
                pltpu.async_copy(onesv, acc.at[colv.at[b, s]], sems[b],
                                 add=True)

        def drain(b):
            for s in range(SUP):
                pltpu.make_async_copy(onesv, acc.at[colv.at[b, s]],
                                      sems[b]).wait()

        pltpu.sync_copy(ei_hbm.at[1, pl.ds(base * SUP, SUP)], colv.at[0])
        fire(0)

        def body(jj, carry):
            for b in range(2):
                nb = 1 - b
                sc = jj * 2 + b
                pltpu.sync_copy(
                    ei_hbm.at[1, pl.ds((base + sc + 1) * SUP, SUP)],
                    colv.at[nb])
                drain(b)
                if b == 0:
                    fire(nb)
                else:
                    @pl.when(jj < nsup // 2 - 1)
                    def _():
                        fire(nb)
            return carry

        lax.fori_loop(0, nsup // 2, body, 0)
        plsc.subcore_barrier()
        off = pl.multiple_of(cid * NPAD + sid * DSTRIPE, 8)
        pltpu.sync_copy(acc.at[pl.ds(sid * DSTRIPE, DSTRIPE)],
                        out_hbm.at[pl.ds(off, DSTRIPE)])

    @functools.partial(
        pl.kernel,
        out_type=[jax.ShapeDtypeStruct((NPAD, D_out), jnp.float32),
                  jax.ShapeDtypeStruct((NPAD, D_out), jnp.float32)],
        mesh=_mesh(),
        scratch_types=[
            pltpu.VMEM((2, SUP, CHUNK), jnp.int32),
            pltpu.VMEM((2, SUP, CHUNK), jnp.int32),
            pltpu.VMEM((2, CHUNK, D_out), jnp.float32),
            [pltpu.SemaphoreType.DMA] * 2,
            [pltpu.SemaphoreType.DMA] * 2,
            pltpu.VMEM_SHARED((NPAD, D_out), jnp.float32),
        ],
    )
    def _agg(hp_hbm, ei_hbm, zrows_hbm, out0_hbm, out1_hbm,
             rowv, colv, datav, gsems, ssems, acc):
        cid = lax.axis_index("c")
        sid = lax.axis_index("s")
        wid = sid * NC + cid
        pltpu.sync_copy(zrows_hbm, acc.at[pl.ds(sid * RSTRIPE, RSTRIPE)])
        plsc.subcore_barrier()

        nsup = KCH // SUP
        base = wid * nsup

        def ld_sup(s, ib):
            pltpu.sync_copy(ei_hbm.at[0, pl.ds((base + s) * SUP, SUP)],
                            rowv.at[ib])
            pltpu.sync_copy(ei_hbm.at[1, pl.ds((base + s) * SUP, SUP)],
                            colv.at[ib])

        def g_fire(ib, c, b):
            pltpu.async_copy(hp_hbm.at[rowv.at[ib, c]], datav.at[b],
                             gsems[b])

        def g_wait(ib, c, b):
            pltpu.make_async_copy(hp_hbm.at[rowv.at[ib, c]], datav.at[b],
                                  gsems[b]).wait()

        def s_fire(ib, c, b):
            pltpu.async_copy(datav.at[b], acc.at[colv.at[ib, c]], ssems[b],
                             add=True)

        def s_wait(ib, c, b):
            pltpu.make_async_copy(datav.at[b], acc.at[colv.at[ib, c]],
                                  ssems[b]).wait()

        ld_sup(0, 0)
        g_fire(0, 0, 0)

        def body(tt, carry):
            for sb in range(2):
                t = tt * 2 + sb
                for c in range(SUP):
                    j = t * SUP + c
                    b = c % 2
                    nb = 1 - b
                    g_wait(sb, c, b)
                    s_fire(sb, c, b)
                    cp = (c + SUP - 1) % SUP
                    ibp = sb if c >= 1 else 1 - sb
                    if sb == 0 and c == 0:
                        @pl.when(tt > 0)
                        def _():
                            s_wait(ibp, cp, nb)
                    else:
                        s_wait(ibp, cp, nb)
                    if c == 0:
                        ld_sup(t + 1, 1 - sb)
                    if c < SUP - 1:
                        g_fire(sb, c + 1, nb)
                    else:
                        g_fire(1 - sb, 0, nb)
            return carry

        lax.fori_loop(0, nsup // 2, body, 0)
        s_wait(1, SUP - 1, 1)
        g_wait(0, 0, 0)
        plsc.subcore_barrier()
        stripe = pl.ds(sid * RSTRIPE, RSTRIPE)

        @pl.when(cid == 0)
        def _():
            pltpu.sync_copy(acc.at[stripe], out0_hbm.at[stripe])

        @pl.when(cid == 1)
        def _():
            pltpu.sync_copy(acc.at[stripe], out1_hbm.at[stripe])

    MBLK = next(m for m in range(1024, 0, -8) if N % m == 0)
    GRID = N // MBLK

    def _lin_body(xb, wb, db, hb):
        dis = lax.rsqrt(1.0 + db[...])
        h = jnp.dot(xb[...], wb[...], preferred_element_type=jnp.float32,
                    precision=lax.Precision.HIGHEST)
        hb[...] = h * dis

    _lin = pl.pallas_call(
        _lin_body,
        grid=(GRID,),
        in_specs=[
            pl.BlockSpec((MBLK, D_in), lambda i: (i, 0)),
            pl.BlockSpec((D_in, D_out), lambda i: (0, 0)),
            pl.BlockSpec((MBLK, 1), lambda i: (i, 0)),
        ],
        out_specs=pl.BlockSpec((MBLK, D_out), lambda i: (i, 0)),
        out_shape=jax.ShapeDtypeStruct((N, D_out), jnp.float32),
    )

    def _fin_body(p0b, p1b, hb, db, bb, ob):
        dis = lax.rsqrt(1.0 + db[...])
        ob[...] = (p0b[...] + p1b[...] + hb[...]) * dis + bb[...]

    _fin = pl.pallas_call(
        _fin_body,
        grid=(GRID,),
        in_specs=[
            pl.BlockSpec((MBLK, D_out), lambda i: (i, 0)),
            pl.BlockSpec((MBLK, D_out), lambda i: (i, 0)),
            pl.BlockSpec((MBLK, D_out), lambda i: (i, 0)),
            pl.BlockSpec((MBLK, 1), lambda i: (i, 0)),
            pl.BlockSpec((1, D_out), lambda i: (0, 0)),
        ],
        out_specs=pl.BlockSpec((MBLK, D_out), lambda i: (i, 0)),
        out_shape=jax.ShapeDtypeStruct((N, D_out), jnp.float32),
    )

    @jax.jit
    def run(x, edge_index, W, b):
        npad_e = NCHUNK * CHUNK - E
        ar = jnp.arange(npad_e, dtype=jnp.int32)
        pad = jnp.stack([(ar * 37) % N, N + ar % (NPAD - N)])
        ei = jnp.concatenate([edge_index.astype(jnp.int32), pad],
                             axis=1).reshape(2, NCHUNK, CHUNK)
        deg_flat = _deg(ei)
        dsum = (deg_flat[:NPAD] + deg_flat[NPAD:]).reshape(NPAD, 1)
        hp = _lin(x, W, dsum)
        zrows = jnp.zeros((RSTRIPE, D_out), jnp.float32)
        p0, p1 = _agg(hp, ei, zrows)
        return _fin(p0, p1, hp, dsum, b.reshape(1, D_out))

    return run


def kernel(x, edge_index, W, b):
    N, D_in = x.shape
    D_out = W.shape[1]
    E = edge_index.shape[1]
    return _build(N, E, D_in, D_out)(x, edge_index, W, b)


# speedup vs baseline: 38.1788x; 1.0125x over previous
import functools

import jax
import jax.numpy as jnp
from jax import lax
from jax.experimental import pallas as pl
from jax.experimental.pallas import tpu as pltpu
from jax.experimental.pallas import tpu_sc as plsc

NC = 2
NS = 16
NW = NC * NS
CHUNK = 128


def _mesh():
    return plsc.VectorSubcoreMesh(
        core_axis_name="c", subcore_axis_name="s", num_cores=NC,
        num_subcores=NS)


@functools.lru_cache(maxsize=None)
def _build(N, E, D_in, D_out):
    NPAD = ((N + 1023) // 1024) * 1024
    KCH = -(-E // (NW * CHUNK))
    KCH = ((KCH + 15) // 16) * 16
    E_PAD = NW * CHUNK * KCH
    SUP = 8
    NCHUNK = E_PAD // CHUNK + SUP
    DSTRIPE = NPAD // NS
    RSTRIPE = NPAD // NS

    @functools.partial(
        pl.kernel,
        out_type=jax.ShapeDtypeStruct((2 * NPAD,), jnp.float32),
        mesh=_mesh(),
        scratch_types=[
            pltpu.VMEM((2, SUP, CHUNK), jnp.int32),
            pltpu.VMEM((CHUNK,), jnp.float32),
            pltpu.VMEM((DSTRIPE,), jnp.float32),
            pltpu.SemaphoreType.DMA,
            pltpu.SemaphoreType.DMA,
            pltpu.VMEM_SHARED((NPAD,), jnp.float32),
        ],
    )
    def _deg(ei_hbm, out_hbm, colv, onesv, zv, sem0, sem1, acc):
        cid = lax.axis_index("c")
        sid = lax.axis_index("s")
        wid = sid * NC + cid
        sems = (sem0, sem1)
        for i in range(CHUNK // 16):
            onesv[pl.ds(i * 16, 16)] = jnp.ones((16,), jnp.float32)

        def zbody(i, carry):
            zv[pl.ds(pl.multiple_of(i * 16, 16), 16)] = jnp.zeros(
                (16,), jnp.float32)
            return carry

        lax.fori_loop(0, DSTRIPE // 16, zbody, 0)
        pltpu.sync_copy(zv, acc.at[pl.ds(sid * DSTRIPE, DSTRIPE)])
        plsc.subcore_barrier()

        nsup = KCH // SUP
        base = wid * nsup

        def fire(b):
            for s in range(SUP):
                pltpu.async_copy(onesv, acc.at[colv.at[b, s]], sems[b],
                                 add=True)

        def drain(b):
            for s in range(SUP):
                pltpu.make_async_copy(onesv, acc.at[colv.at[b, s]],
                                      sems[b]).wait()

        pltpu.sync_copy(ei_hbm.at[1, pl.ds(base * SUP, SUP)], colv.at[0])
        fire(0)

        def body(jj, carry):
            for b in range(2):
                nb = 1 - b
                sc = jj * 2 + b
                pltpu.sync_copy(
                    ei_hbm.at[1, pl.ds((base + sc + 1) * SUP, SUP)],
                    colv.at[nb])
                drain(b)
                if b == 0:
                    fire(nb)
                else:
                    @pl.when(jj < nsup // 2 - 1)
                    def _():
                        fire(nb)
            return carry

        lax.fori_loop(0, nsup // 2, body, 0)
        plsc.subcore_barrier()
        off = pl.multiple_of(cid * NPAD + sid * DSTRIPE, 8)
        pltpu.sync_copy(acc.at[pl.ds(sid * DSTRIPE, DSTRIPE)],
                        out_hbm.at[pl.ds(off, DSTRIPE)])

    @functools.partial(
        pl.kernel,
        out_type=[jax.ShapeDtypeStruct((NPAD, D_out), jnp.float32),
                  jax.ShapeDtypeStruct((NPAD, D_out), jnp.float32)],
        mesh=_mesh(),
        scratch_types=[
            pltpu.VMEM((2, SUP, CHUNK), jnp.int32),
            pltpu.VMEM((2, SUP, CHUNK), jnp.int32),
            pltpu.VMEM((2, CHUNK, D_out), jnp.float32),
            [pltpu.SemaphoreType.DMA] * 2,
            [pltpu.SemaphoreType.DMA] * 2,
            pltpu.SemaphoreType.DMA,
            pltpu.VMEM_SHARED((NPAD, D_out), jnp.float32),
        ],
    )
    def _agg(hp_hbm, ei_hbm, zrows_hbm, out0_hbm, out1_hbm,
             rowv, colv, datav, gsems, ssems, zsem, acc):
        cid = lax.axis_index("c")
        sid = lax.axis_index("s")
        wid = sid * NC + cid
        zcopy = pltpu.async_copy(
            zrows_hbm, acc.at[pl.ds(sid * RSTRIPE, RSTRIPE)], zsem)

        nsup = KCH // SUP
        base = wid * nsup

        def ld_sup(s, ib):
            pltpu.sync_copy(ei_hbm.at[0, pl.ds((base + s) * SUP, SUP)],
                            rowv.at[ib])
            pltpu.sync_copy(ei_hbm.at[1, pl.ds((base + s) * SUP, SUP)],
                            colv.at[ib])

        def g_fire(ib, c, b):
            pltpu.async_copy(hp_hbm.at[rowv.at[ib, c]], datav.at[b],
                             gsems[b])

        def g_wait(ib, c, b):
            pltpu.make_async_copy(hp_hbm.at[rowv.at[ib, c]], datav.at[b],
                                  gsems[b]).wait()

        def s_fire(ib, c, b):
            pltpu.async_copy(datav.at[b], acc.at[colv.at[ib, c]], ssems[b],
                             add=True)

        def s_wait(ib, c, b):
            pltpu.make_async_copy(datav.at[b], acc.at[colv.at[ib, c]],
                                  ssems[b]).wait()

        ld_sup(0, 0)
        g_fire(0, 0, 0)
        zcopy.wait()
        plsc.subcore_barrier()

        def body(tt, carry):
            for sb in range(2):
                t = tt * 2 + sb
                for c in range(SUP):
                    j = t * SUP + c
                    b = c % 2
                    nb = 1 - b
                    g_wait(sb, c, b)
                    s_fire(sb, c, b)
                    cp = (c + SUP - 1) % SUP
                    ibp = sb if c >= 1 else 1 - sb
                    if sb == 0 and c == 0:
                        @pl.when(tt > 0)
                        def _():
                            s_wait(ibp, cp, nb)
                    else:
                        s_wait(ibp, cp, nb)
                    if c == 0:
                        ld_sup(t + 1, 1 - sb)
                    if c < SUP - 1:
                        g_fire(sb, c + 1, nb)
                    else:
                        g_fire(1 - sb, 0, nb)
            return carry

        lax.fori_loop(0, nsup // 2, body, 0)
        s_wait(1, SUP - 1, 1)
        g_wait(0, 0, 0)
        plsc.subcore_barrier()
        stripe = pl.ds(sid * RSTRIPE, RSTRIPE)

        @pl.when(cid == 0)
        def _():
            pltpu.sync_copy(acc.at[stripe], out0_hbm.at[stripe])

        @pl.when(cid == 1)
        def _():
            pltpu.sync_copy(acc.at[stripe], out1_hbm.at[stripe])

    MBLK = next(m for m in range(1024, 0, -8) if N % m == 0)
    GRID = N // MBLK

    def _lin_body(xb, wb, db, hb):
        dis = lax.rsqrt(1.0 + db[...])
        h = jnp.dot(xb[...], wb[...], preferred_element_type=jnp.float32,
                    precision=lax.Precision.HIGHEST)
        hb[...] = h * dis

    _lin = pl.pallas_call(
        _lin_body,
        grid=(GRID,),
        in_specs=[
            pl.BlockSpec((MBLK, D_in), lambda i: (i, 0)),
            pl.BlockSpec((D_in, D_out), lambda i: (0, 0)),
            pl.BlockSpec((MBLK, 1), lambda i: (i, 0)),
        ],
        out_specs=pl.BlockSpec((MBLK, D_out), lambda i: (i, 0)),
        out_shape=jax.ShapeDtypeStruct((N, D_out), jnp.float32),
    )

    def _fin_body(p0b, p1b, hb, db, bb, ob):
        dis = lax.rsqrt(1.0 + db[...])
        ob[...] = (p0b[...] + p1b[...] + hb[...]) * dis + bb[...]

    _fin = pl.pallas_call(
        _fin_body,
        grid=(GRID,),
        in_specs=[
            pl.BlockSpec((MBLK, D_out), lambda i: (i, 0)),
            pl.BlockSpec((MBLK, D_out), lambda i: (i, 0)),
            pl.BlockSpec((MBLK, D_out), lambda i: (i, 0)),
            pl.BlockSpec((MBLK, 1), lambda i: (i, 0)),
            pl.BlockSpec((1, D_out), lambda i: (0, 0)),
        ],
        out_specs=pl.BlockSpec((MBLK, D_out), lambda i: (i, 0)),
        out_shape=jax.ShapeDtypeStruct((N, D_out), jnp.float32),
    )

    @jax.jit
    def run(x, edge_index, W, b):
        npad_e = NCHUNK * CHUNK - E
        ar = jnp.arange(npad_e, dtype=jnp.int32)
        pad = jnp.stack([(ar * 37) % N, N + ar % (NPAD - N)])
        ei = jnp.concatenate([edge_index.astype(jnp.int32), pad],
                             axis=1).reshape(2, NCHUNK, CHUNK)
        deg_flat = _deg(ei)
        dsum = (deg_flat[:NPAD] + deg_flat[NPAD:]).reshape(NPAD, 1)
        hp = _lin(x, W, dsum)
        zrows = jnp.zeros((RSTRIPE, D_out), jnp.float32)
        p0, p1 = _agg(hp, ei, zrows)
        return _fin(p0, p1, hp, dsum, b.reshape(1, D_out))

    return run


def kernel(x, edge_index, W, b):
    N, D_in = x.shape
    D_out = W.shape[1]
    E = edge_index.shape[1]
    return _build(N, E, D_in, D_out)(x, edge_index, W, b)


# speedup vs baseline: 40.2122x; 1.0533x over previous
import functools

import jax
import jax.numpy as jnp
from jax import lax
from jax.experimental import pallas as pl
from jax.experimental.pallas import tpu as pltpu
from jax.experimental.pallas import tpu_sc as plsc

NC = 2
NS = 16
NW = NC * NS
CHUNK = 128


def _mesh():
    return plsc.VectorSubcoreMesh(
        core_axis_name="c", subcore_axis_name="s", num_cores=NC,
        num_subcores=NS)


@functools.lru_cache(maxsize=None)
def _build(N, E, D_in, D_out):
    NPAD = ((N + 1023) // 1024) * 1024
    KCH = -(-E // (NW * CHUNK))
    KCH = ((KCH + 15) // 16) * 16
    E_PAD = NW * CHUNK * KCH
    SUP = 8
    NCHUNK = E_PAD // CHUNK + SUP
    DSTRIPE = NPAD // NS
    RSTRIPE = NPAD // NS

    @functools.partial(
        pl.kernel,
        out_type=jax.ShapeDtypeStruct((2 * NPAD,), jnp.float32),
        mesh=_mesh(),
        scratch_types=[
            pltpu.VMEM((2, SUP, CHUNK), jnp.int32),
            pltpu.VMEM((CHUNK,), jnp.float32),
            pltpu.VMEM((DSTRIPE,), jnp.float32),
            pltpu.SemaphoreType.DMA,
            pltpu.SemaphoreType.DMA,
            pltpu.VMEM_SHARED((NPAD,), jnp.float32),
        ],
    )
    def _deg(ei_hbm, out_hbm, colv, onesv, zv, sem0, sem1, acc):
        cid = lax.axis_index("c")
        sid = lax.axis_index("s")
        wid = sid * NC + cid
        sems = (sem0, sem1)
        for i in range(CHUNK // 16):
            onesv[pl.ds(i * 16, 16)] = jnp.ones((16,), jnp.float32)

        def zbody(i, carry):
            zv[pl.ds(pl.multiple_of(i * 16, 16), 16)] = jnp.zeros(
                (16,), jnp.float32)
            return carry

        lax.fori_loop(0, DSTRIPE // 16, zbody, 0)
        pltpu.sync_copy(zv, acc.at[pl.ds(sid * DSTRIPE, DSTRIPE)])
        plsc.subcore_barrier()

        nsup = KCH // SUP
        base = wid * nsup

        def fire(b):
            for s in range(SUP):
                pltpu.async_copy(onesv, acc.at[colv.at[b, s]], sems[b],
                                 add=True)

        def drain(b):
            for s in range(SUP):
                pltpu.make_async_copy(onesv, acc.at[colv.at[b, s]],
                                      sems[b]).wait()

        pltpu.sync_copy(ei_hbm.at[1, pl.ds(base * SUP, SUP)], colv.at[0])
        fire(0)

        def body(jj, carry):
            for b in range(2):
                nb = 1 - b
                sc = jj * 2 + b
                pltpu.sync_copy(
                    ei_hbm.at[1, pl.ds((base + sc + 1) * SUP, SUP)],
                    colv.at[nb])
                drain(b)
                if b == 0:
                    fire(nb)
                else:
                    @pl.when(jj < nsup // 2 - 1)
                    def _():
                        fire(nb)
            return carry

        lax.fori_loop(0, nsup // 2, body, 0)
        plsc.subcore_barrier()
        off = pl.multiple_of(cid * NPAD + sid * DSTRIPE, 8)
        pltpu.sync_copy(acc.at[pl.ds(sid * DSTRIPE, DSTRIPE)],
                        out_hbm.at[pl.ds(off, DSTRIPE)])

    @functools.partial(
        pl.kernel,
        out_type=[jax.ShapeDtypeStruct((NPAD, D_out), jnp.float32),
                  jax.ShapeDtypeStruct((NPAD, D_out), jnp.float32)],
        mesh=_mesh(),
        scratch_types=[
            pltpu.VMEM((2, SUP, CHUNK), jnp.int32),
            pltpu.VMEM((2, SUP, CHUNK), jnp.int32),
            pltpu.VMEM((2, CHUNK, D_out), jnp.float32),
            [pltpu.SemaphoreType.DMA] * 2,
            [pltpu.SemaphoreType.DMA] * 2,
            pltpu.SemaphoreType.DMA,
            pltpu.SemaphoreType.DMA,
            pltpu.VMEM_SHARED((NPAD, D_out), jnp.float32),
        ],
    )
    def _agg(hp_hbm, ei_hbm, zrows_hbm, out0_hbm, out1_hbm,
             rowv, colv, datav, gsems, ssems, zsem, isem, acc):
        cid = lax.axis_index("c")
        sid = lax.axis_index("s")
        wid = sid * NC + cid
        zcopy = pltpu.async_copy(
            zrows_hbm, acc.at[pl.ds(sid * RSTRIPE, RSTRIPE)], zsem)

        nsup = KCH // SUP
        base = wid * nsup

        def ld_sup(s, ib):
            pltpu.sync_copy(ei_hbm.at[0, pl.ds((base + s) * SUP, SUP)],
                            rowv.at[ib])
            pltpu.sync_copy(ei_hbm.at[1, pl.ds((base + s) * SUP, SUP)],
                            colv.at[ib])

        def ld_sup_fire(s, ib):
            pltpu.async_copy(ei_hbm.at[0, pl.ds((base + s) * SUP, SUP)],
                             rowv.at[ib], isem)
            pltpu.async_copy(ei_hbm.at[1, pl.ds((base + s) * SUP, SUP)],
                             colv.at[ib], isem)

        def ld_sup_wait(s, ib):
            pltpu.make_async_copy(ei_hbm.at[0, pl.ds((base + s) * SUP, SUP)],
                                  rowv.at[ib], isem).wait()
            pltpu.make_async_copy(ei_hbm.at[1, pl.ds((base + s) * SUP, SUP)],
                                  colv.at[ib], isem).wait()

        def g_fire(ib, c, b):
            pltpu.async_copy(hp_hbm.at[rowv.at[ib, c]], datav.at[b],
                             gsems[b])

        def g_wait(ib, c, b):
            pltpu.make_async_copy(hp_hbm.at[rowv.at[ib, c]], datav.at[b],
                                  gsems[b]).wait()

        def s_fire(ib, c, b):
            pltpu.async_copy(datav.at[b], acc.at[colv.at[ib, c]], ssems[b],
                             add=True)

        def s_wait(ib, c, b):
            pltpu.make_async_copy(datav.at[b], acc.at[colv.at[ib, c]],
                                  ssems[b]).wait()

        ld_sup(0, 0)
        g_fire(0, 0, 0)
        zcopy.wait()
        plsc.subcore_barrier()

        def body(tt, carry):
            for sb in range(2):
                t = tt * 2 + sb
                for c in range(SUP):
                    j = t * SUP + c
                    b = c % 2
                    nb = 1 - b
                    g_wait(sb, c, b)
                    s_fire(sb, c, b)
                    cp = (c + SUP - 1) % SUP
                    ibp = sb if c >= 1 else 1 - sb
                    if sb == 0 and c == 0:
                        @pl.when(tt > 0)
                        def _():
                            s_wait(ibp, cp, nb)
                    else:
                        s_wait(ibp, cp, nb)
                    if c == 0:
                        ld_sup_fire(t + 1, 1 - sb)
                    if c < SUP - 1:
                        g_fire(sb, c + 1, nb)
                    else:
                        ld_sup_wait(t + 1, 1 - sb)
                        g_fire(1 - sb, 0, nb)
            return carry

        lax.fori_loop(0, nsup // 2, body, 0)
        s_wait(1, SUP - 1, 1)
        g_wait(0, 0, 0)
        plsc.subcore_barrier()
        stripe = pl.ds(sid * RSTRIPE, RSTRIPE)

        @pl.when(cid == 0)
        def _():
            pltpu.sync_copy(acc.at[stripe], out0_hbm.at[stripe])

        @pl.when(cid == 1)
        def _():
            pltpu.sync_copy(acc.at[stripe], out1_hbm.at[stripe])

    MBLK = next(m for m in range(1024, 0, -8) if N % m == 0)
    GRID = N // MBLK

    def _lin_body(xb, wb, db, hb):
        dis = lax.rsqrt(1.0 + db[...])
        h = jnp.dot(xb[...], wb[...], preferred_element_type=jnp.float32,
                    precision=lax.Precision.HIGHEST)
        hb[...] = h * dis

    _lin = pl.pallas_call(
        _lin_body,
        grid=(GRID,),
        in_specs=[
            pl.BlockSpec((MBLK, D_in), lambda i: (i, 0)),
            pl.BlockSpec((D_in, D_out), lambda i: (0, 0)),
            pl.BlockSpec((MBLK, 1), lambda i: (i, 0)),
        ],
        out_specs=pl.BlockSpec((MBLK, D_out), lambda i: (i, 0)),
        out_shape=jax.ShapeDtypeStruct((N, D_out), jnp.float32),
    )

    def _fin_body(p0b, p1b, hb, db, bb, ob):
        dis = lax.rsqrt(1.0 + db[...])
        ob[...] = (p0b[...] + p1b[...] + hb[...]) * dis + bb[...]

    _fin = pl.pallas_call(
        _fin_body,
        grid=(GRID,),
        in_specs=[
            pl.BlockSpec((MBLK, D_out), lambda i: (i, 0)),
            pl.BlockSpec((MBLK, D_out), lambda i: (i, 0)),
            pl.BlockSpec((MBLK, D_out), lambda i: (i, 0)),
            pl.BlockSpec((MBLK, 1), lambda i: (i, 0)),
            pl.BlockSpec((1, D_out), lambda i: (0, 0)),
        ],
        out_specs=pl.BlockSpec((MBLK, D_out), lambda i: (i, 0)),
        out_shape=jax.ShapeDtypeStruct((N, D_out), jnp.float32),
    )

    @jax.jit
    def run(x, edge_index, W, b):
        npad_e = NCHUNK * CHUNK - E
        ar = jnp.arange(npad_e, dtype=jnp.int32)
        pad = jnp.stack([(ar * 37) % N, N + ar % (NPAD - N)])
        ei = jnp.concatenate([edge_index.astype(jnp.int32), pad],
                             axis=1).reshape(2, NCHUNK, CHUNK)
        deg_flat = _deg(ei)
        dsum = (deg_flat[:NPAD] + deg_flat[NPAD:]).reshape(NPAD, 1)
        hp = _lin(x, W, dsum)
        zrows = jnp.zeros((RSTRIPE, D_out), jnp.float32)
        p0, p1 = _agg(hp, ei, zrows)
        return _fin(p0, p1, hp, dsum, b.reshape(1, D_out))

    return run


def kernel(x, edge_index, W, b):
    N, D_in = x.shape
    D_out = W.shape[1]
    E = edge_index.shape[1]
    return _build(N, E, D_in, D_out)(x, edge_index, W, b)
